# Initial kernel scaffold; baseline (speedup 1.0000x reference)
import jax
import jax.numpy as jnp
from jax.experimental import pallas as pl


def kernel(x, logits):
    raise NotImplementedError("write your pallas kernel here")


# speedup vs baseline: 5.1171x; 5.1171x over previous
import functools

import jax
import jax.numpy as jnp
import numpy as np
from jax.experimental import pallas as pl

B = 16384
V = 27
JPAD = 32

_U32 = jnp.uint32
_K1 = np.uint32(0)
_K2 = np.uint32(42)
_K3 = np.uint32(0 ^ 42 ^ 0x1BD11BDA)
_TINY = np.float32(np.finfo(np.float32).tiny)


def _rotl(x, r):
    return (x << _U32(r)) | (x >> _U32(32 - r))


def _threefry_bits(n):
    rotations = ((13, 15, 26, 6), (17, 29, 16, 24))
    ks = (_K1, _K2, _K3)
    x0 = jnp.zeros_like(n) + ks[0]
    x1 = n + ks[1]
    for i in range(5):
        for r in rotations[i % 2]:
            x0 = x0 + x1
            x1 = _rotl(x1, r)
            x1 = x0 ^ x1
        x0 = x0 + ks[(i + 1) % 3]
        x1 = x1 + ks[(i + 2) % 3] + _U32(i + 1)
    return x0 ^ x1


def _gumbel_from_bits(bits):
    fb = (bits >> _U32(9)) | _U32(0x3F800000)
    f = jax.lax.bitcast_convert_type(fb, jnp.float32) - jnp.float32(1.0)
    u = f * (jnp.float32(1.0) - _TINY) + _TINY
    u = jnp.maximum(_TINY, u)
    return -jnp.log(-jnp.log(u))


def _body(x_ref, lt_ref, out_ref):
    j = jax.lax.broadcasted_iota(jnp.int32, (JPAD, B), 0)
    i = jax.lax.broadcasted_iota(jnp.int32, (JPAD, B), 1)
    n = (i * V + j).astype(_U32)
    g = _gumbel_from_bits(_threefry_bits(n))

    tab = jnp.log(lt_ref[...])
    onehot = (j == x_ref[...]).astype(jnp.float32)
    logp = jax.lax.dot_general(
        tab, onehot, (((1,), (0,)), ((), ())),
        preferred_element_type=jnp.float32,
        precision=jax.lax.Precision.HIGHEST)

    scores = g + logp
    scores = jnp.where(j < V, scores, -jnp.inf)

    val, idx = scores, j
    for size in (16, 8, 4, 2, 1):
        av, bv = val[:size], val[size:2 * size]
        ai, bi = idx[:size], idx[size:2 * size]
        takeb = (bv > av) | ((bv == av) & (bi < ai))
        val = jnp.where(takeb, bv, av)
        idx = jnp.where(takeb, bi, ai)
    out_ref[...] = idx


@functools.partial(jax.jit, static_argnames=())
def kernel(x, logits):
    xr = x.reshape(1, B).astype(jnp.int32)
    lt = jnp.ones((JPAD, JPAD), jnp.float32).at[:V, :V].set(logits.T)
    out = pl.pallas_call(
        _body,
        out_shape=jax.ShapeDtypeStruct((1, B), jnp.int32),
    )(xr, lt)
    return out.reshape(B, 1)
